# Initial kernel scaffold; baseline (speedup 1.0000x reference)
import jax
import jax.numpy as jnp
from jax.experimental import pallas as pl


def kernel(query, prompt_key, prompts, frequency):
    raise NotImplementedError("write your pallas kernel here")


# speedup vs baseline: 3.0336x; 3.0336x over previous
import functools

import jax
import jax.numpy as jnp
from jax import lax
from jax.experimental import pallas as pl
from jax.experimental.pallas import tpu as pltpu
from jax.experimental.pallas import tpu_sc as plsc

_POOL = 20
_SEL = 10
_PLEN = 8
_DIM = 1024
_B = 1024


def _rowsum(s):
    acc = s[:, 0:128]
    for i in range(1, 8):
        acc = acc + s[:, i * 128:(i + 1) * 128]
    u = acc[:, 0:8]
    for k in range(1, 16):
        u = u + acc[:, 8 * k:8 * (k + 1)]
    u = u[:, 0:4] + u[:, 4:8]
    u = u[:, 0:2] + u[:, 2:4]
    return u[:, 0:1] + u[:, 1:2]


def _selector_body(q_ref, pk_ref, freq_ref, sim_ref, unsim_ref, flat_ref):
    q = q_ref[...]
    pk = pk_ref[...]
    freq = freq_ref[...]

    dot = lax.dot_general(
        q, pk, (((1,), (1,)), ((), ())),
        preferred_element_type=jnp.float32,
    )
    qn = jnp.sqrt(_rowsum(q * q))
    kn = jnp.sqrt(_rowsum(pk * pk)).reshape(1, _POOL)
    match = dot / jnp.maximum(qn * kn, 1e-8)

    fr = 1.0 / freq
    scale = fr / jnp.maximum(jnp.sum(jnp.abs(fr)), 1e-12)
    scores = match * scale

    cols = lax.broadcasted_iota(jnp.int32, (1, _POOL), 1)

    member_cols = []
    for p in range(_POOL):
        sp = scores[:, p:p + 1]
        beat = (scores > sp) | ((scores == sp) & (cols < p))
        rank_p = jnp.sum(beat.astype(jnp.int32), axis=1, keepdims=True)
        member_cols.append((rank_p < _SEL).astype(jnp.int32))
    member = jnp.concatenate(member_cols, axis=1)
    counts = jnp.sum(member, axis=0, keepdims=True)
    masked = jnp.where(counts > 0, counts, -1)

    r_cols = []
    for p in range(_POOL):
        mp = masked[:, p:p + 1]
        beat = (masked > mp) | ((masked == mp) & (cols < p))
        r_cols.append(jnp.sum(beat.astype(jnp.int32), axis=1, keepdims=True))
    r = jnp.concatenate(r_cols, axis=1)
    selected = r < _SEL

    u_cols = []
    for p in range(_POOL):
        below = ((cols < p) & jnp.logical_not(selected)).astype(jnp.int32)
        u_cols.append(jnp.sum(below, axis=1, keepdims=True))
    u = jnp.concatenate(u_cols, axis=1)

    sim_cols, unsim_cols, selrow_cols = [], [], []
    notsel = jnp.logical_not(selected)
    for j in range(_SEL):
        mj = (selected & (r == j))
        mjf = mj.astype(jnp.float32)
        sim_cols.append(jnp.sum(match * mjf, axis=1, keepdims=True))
        uj = (notsel & (u == j)).astype(jnp.float32)
        unsim_cols.append(jnp.sum(match * uj, axis=1, keepdims=True))
        selrow_cols.append(jnp.sum(mj.astype(jnp.int32) * cols,
                                   axis=1, keepdims=True))
    sim_ref[...] = jnp.concatenate(sim_cols, axis=1)
    unsim_ref[...] = jnp.concatenate(unsim_cols, axis=1)

    flat_cols = []
    for j in range(_SEL):
        for l in range(_PLEN):
            flat_cols.append(selrow_cols[j] * _PLEN + l)
    flat_ref[...] = jnp.concatenate(flat_cols, axis=1)


@jax.jit
def _selector_call(query, prompt_key, freq2d):
    return pl.pallas_call(
        _selector_body,
        out_shape=(
            jax.ShapeDtypeStruct((_B, _SEL), jnp.float32),
            jax.ShapeDtypeStruct((_B, _SEL), jnp.float32),
            jax.ShapeDtypeStruct((1, _SEL * _PLEN), jnp.int32),
        ),
    )(query, prompt_key, freq2d)


_NROWS = _SEL * _PLEN


def _make_bcast():
    info = plsc.get_sparse_core_info()
    nw = info.num_cores * info.num_subcores
    rpw = _B // nw
    mesh = plsc.VectorSubcoreMesh(core_axis_name="c", subcore_axis_name="s")

    @functools.partial(
        pl.kernel,
        mesh=mesh,
        out_type=jax.ShapeDtypeStruct((_B, _NROWS, _DIM), jnp.float32),
        scratch_types=[
            pltpu.VMEM((_NROWS,), jnp.int32),
            pltpu.VMEM((_NROWS, _DIM), jnp.float32),
            pltpu.SemaphoreType.DMA,
            pltpu.SemaphoreType.DMA,
        ],
    )
    def bcast(flat_hbm, prompts_hbm, out_hbm, idx_v, rows_v, gsem, wsem):
        wid = lax.axis_index("s") * info.num_cores + lax.axis_index("c")
        base = wid * rpw
        pltpu.sync_copy(flat_hbm, idx_v)
        pltpu.async_copy(prompts_hbm.at[idx_v], rows_v, gsem).wait()
        copies = [pltpu.async_copy(rows_v, out_hbm.at[base + k], wsem)
                  for k in range(rpw)]
        for c in copies:
            c.wait()

    return bcast


_bcast_cache = []


def kernel(query, prompt_key, prompts, frequency):
    if not _bcast_cache:
        _bcast_cache.append(_make_bcast())
    sim, unsim, flat2d = _selector_call(query, prompt_key,
                                        frequency.reshape(1, _POOL))
    flat = flat2d.reshape(_NROWS)
    sel_big = _bcast_cache[0](flat, prompts.reshape(_POOL * _PLEN, _DIM))
    selection = sel_big.reshape(_B, _SEL, _PLEN, _DIM)
    return sim, unsim, selection


# speedup vs baseline: 3.0870x; 1.0176x over previous
import functools

import jax
import jax.numpy as jnp
from jax import lax
from jax.experimental import pallas as pl
from jax.experimental.pallas import tpu as pltpu
from jax.experimental.pallas import tpu_sc as plsc

_POOL = 20
_SEL = 10
_PLEN = 8
_DIM = 1024
_B = 1024


def _rowsum(s):
    acc = s[:, 0:128]
    for i in range(1, 8):
        acc = acc + s[:, i * 128:(i + 1) * 128]
    u = acc[:, 0:8]
    for k in range(1, 16):
        u = u + acc[:, 8 * k:8 * (k + 1)]
    u = u[:, 0:4] + u[:, 4:8]
    u = u[:, 0:2] + u[:, 2:4]
    return u[:, 0:1] + u[:, 1:2]


def _selector_body(q_ref, pk_ref, freq_ref, sim_ref, unsim_ref, flat_ref):
    q = q_ref[...]
    pk = pk_ref[...]
    freq = freq_ref[...]

    dot = lax.dot_general(
        q, pk, (((1,), (1,)), ((), ())),
        preferred_element_type=jnp.float32,
    )
    qn = jnp.sqrt(_rowsum(q * q))
    kn = jnp.sqrt(_rowsum(pk * pk)).reshape(1, _POOL)
    match = dot / jnp.maximum(qn * kn, 1e-8)

    fr = 1.0 / freq
    scale = fr / jnp.maximum(jnp.sum(jnp.abs(fr)), 1e-12)
    scores = match * scale

    cols = lax.broadcasted_iota(jnp.int32, (1, _POOL), 1)

    rank = jnp.zeros((_B, _POOL), jnp.int32)
    for j in range(_POOL):
        sj = scores[:, j:j + 1]
        beat = (sj > scores) | ((sj == scores) & (cols > j))
        rank = rank + beat.astype(jnp.int32)
    member = (rank < _SEL).astype(jnp.int32)
    counts = jnp.sum(member, axis=0, keepdims=True)
    masked = jnp.where(counts > 0, counts, -1)

    r_cols = []
    for p in range(_POOL):
        mp = masked[:, p:p + 1]
        beat = (masked > mp) | ((masked == mp) & (cols < p))
        r_cols.append(jnp.sum(beat.astype(jnp.int32), axis=1, keepdims=True))
    r = jnp.concatenate(r_cols, axis=1)
    selected = r < _SEL

    u_cols = []
    for p in range(_POOL):
        below = ((cols < p) & jnp.logical_not(selected)).astype(jnp.int32)
        u_cols.append(jnp.sum(below, axis=1, keepdims=True))
    u = jnp.concatenate(u_cols, axis=1)

    jcols = lax.broadcasted_iota(jnp.int32, (1, _SEL), 1)
    notsel = jnp.logical_not(selected)
    sim = jnp.zeros((_B, _SEL), jnp.float32)
    unsim = jnp.zeros((_B, _SEL), jnp.float32)
    selrow_cols = []
    for p in range(_POOL):
        mcol = match[:, p:p + 1]
        onehot_s = (selected[:, p:p + 1] & (r[:, p:p + 1] == jcols))
        sim = sim + mcol * onehot_s.astype(jnp.float32)
        onehot_u = (notsel[:, p:p + 1] & (u[:, p:p + 1] == jcols))
        unsim = unsim + mcol * onehot_u.astype(jnp.float32)
    for j in range(_SEL):
        mj = (selected & (r == j))
        selrow_cols.append(jnp.sum(mj.astype(jnp.int32) * cols,
                                   axis=1, keepdims=True))
    sim_ref[...] = sim
    unsim_ref[...] = unsim

    flat_cols = []
    for j in range(_SEL):
        for l in range(_PLEN):
            flat_cols.append(selrow_cols[j] * _PLEN + l)
    flat_ref[...] = jnp.concatenate(flat_cols, axis=1)


@jax.jit
def _selector_call(query, prompt_key, freq2d):
    return pl.pallas_call(
        _selector_body,
        out_shape=(
            jax.ShapeDtypeStruct((_B, _SEL), jnp.float32),
            jax.ShapeDtypeStruct((_B, _SEL), jnp.float32),
            jax.ShapeDtypeStruct((1, _SEL * _PLEN), jnp.int32),
        ),
    )(query, prompt_key, freq2d)


_NROWS = _SEL * _PLEN


def _make_bcast():
    info = plsc.get_sparse_core_info()
    nw = info.num_cores * info.num_subcores
    rpw = _B // nw
    mesh = plsc.VectorSubcoreMesh(core_axis_name="c", subcore_axis_name="s")

    @functools.partial(
        pl.kernel,
        mesh=mesh,
        out_type=jax.ShapeDtypeStruct((_B, _NROWS, _DIM), jnp.float32),
        scratch_types=[
            pltpu.VMEM((_NROWS,), jnp.int32),
            pltpu.VMEM((_NROWS, _DIM), jnp.float32),
            pltpu.SemaphoreType.DMA,
            pltpu.SemaphoreType.DMA,
        ],
    )
    def bcast(flat_hbm, prompts_hbm, out_hbm, idx_v, rows_v, gsem, wsem):
        wid = lax.axis_index("s") * info.num_cores + lax.axis_index("c")
        base = wid * rpw
        pltpu.sync_copy(flat_hbm, idx_v)
        pltpu.async_copy(prompts_hbm.at[idx_v], rows_v, gsem).wait()
        copies = [pltpu.async_copy(rows_v, out_hbm.at[base + k], wsem)
                  for k in range(rpw)]
        for c in copies:
            c.wait()

    return bcast


_bcast_cache = []


def kernel(query, prompt_key, prompts, frequency):
    if not _bcast_cache:
        _bcast_cache.append(_make_bcast())
    sim, unsim, flat2d = _selector_call(query, prompt_key,
                                        frequency.reshape(1, _POOL))
    flat = flat2d.reshape(_NROWS)
    sel_big = _bcast_cache[0](flat, prompts.reshape(_POOL * _PLEN, _DIM))
    selection = sel_big.reshape(_B, _SEL, _PLEN, _DIM)
    return sim, unsim, selection


# speedup vs baseline: 3.1209x; 1.0110x over previous
import functools

import jax
import jax.numpy as jnp
from jax import lax
from jax.experimental import pallas as pl
from jax.experimental.pallas import tpu as pltpu
from jax.experimental.pallas import tpu_sc as plsc

_POOL = 20
_SEL = 10
_PLEN = 8
_DIM = 1024
_B = 1024


def _rowsum(s):
    acc = s[:, 0:128]
    for i in range(1, 8):
        acc = acc + s[:, i * 128:(i + 1) * 128]
    u = acc[:, 0:8]
    for k in range(1, 16):
        u = u + acc[:, 8 * k:8 * (k + 1)]
    u = u[:, 0:4] + u[:, 4:8]
    u = u[:, 0:2] + u[:, 2:4]
    return u[:, 0:1] + u[:, 1:2]


def _selector_body(q_ref, pk_ref, freq_ref, sim_ref, unsim_ref, flat_ref):
    q = q_ref[...]
    pk = pk_ref[...]
    freq = freq_ref[...]

    dot = lax.dot_general(
        q, pk, (((1,), (1,)), ((), ())),
        preferred_element_type=jnp.float32,
    )
    qn = jnp.sqrt(_rowsum(q * q))
    kn = jnp.sqrt(_rowsum(pk * pk)).reshape(1, _POOL)
    match = dot / jnp.maximum(qn * kn, 1e-8)

    fr = 1.0 / freq
    scale = fr / jnp.maximum(jnp.sum(jnp.abs(fr)), 1e-12)
    scores = match * scale

    cols = lax.broadcasted_iota(jnp.int32, (1, _POOL), 1)

    rank = jnp.zeros((_B, _POOL), jnp.int32)
    for j in range(_POOL):
        sj = scores[:, j:j + 1]
        beat = (sj > scores) | ((sj == scores) & (cols > j))
        rank = rank + beat.astype(jnp.int32)
    member = (rank < _SEL).astype(jnp.int32)
    counts = jnp.sum(member, axis=0, keepdims=True)
    masked = jnp.where(counts > 0, counts, -1)

    r_cols = []
    for p in range(_POOL):
        mp = masked[:, p:p + 1]
        beat = (masked > mp) | ((masked == mp) & (cols < p))
        r_cols.append(jnp.sum(beat.astype(jnp.int32), axis=1, keepdims=True))
    r = jnp.concatenate(r_cols, axis=1)
    selected = r < _SEL

    u_cols = []
    for p in range(_POOL):
        below = ((cols < p) & jnp.logical_not(selected)).astype(jnp.int32)
        u_cols.append(jnp.sum(below, axis=1, keepdims=True))
    u = jnp.concatenate(u_cols, axis=1)

    notsel = jnp.logical_not(selected)
    pos20 = jnp.where(selected, r, u + _SEL)
    y = jnp.zeros((_B, _POOL), jnp.float32)
    for p in range(_POOL):
        onehot = (pos20[:, p:p + 1] == cols).astype(jnp.float32)
        y = y + match[:, p:p + 1] * onehot
    selrow_cols = []
    for j in range(_SEL):
        mj = (selected & (r == j))
        selrow_cols.append(jnp.sum(mj.astype(jnp.int32) * cols,
                                   axis=1, keepdims=True))
    sim_ref[...] = y[:, 0:_SEL]
    unsim_ref[...] = y[:, _SEL:_POOL]

    flat_cols = []
    for j in range(_SEL):
        for l in range(_PLEN):
            flat_cols.append(selrow_cols[j] * _PLEN + l)
    flat_ref[...] = jnp.concatenate(flat_cols, axis=1)


@jax.jit
def _selector_call(query, prompt_key, freq2d):
    return pl.pallas_call(
        _selector_body,
        out_shape=(
            jax.ShapeDtypeStruct((_B, _SEL), jnp.float32),
            jax.ShapeDtypeStruct((_B, _SEL), jnp.float32),
            jax.ShapeDtypeStruct((1, _SEL * _PLEN), jnp.int32),
        ),
    )(query, prompt_key, freq2d)


_NROWS = _SEL * _PLEN


def _make_bcast():
    info = plsc.get_sparse_core_info()
    nw = info.num_cores * info.num_subcores
    rpw = _B // nw
    mesh = plsc.VectorSubcoreMesh(core_axis_name="c", subcore_axis_name="s")

    @functools.partial(
        pl.kernel,
        mesh=mesh,
        out_type=jax.ShapeDtypeStruct((_B, _NROWS, _DIM), jnp.float32),
        scratch_types=[
            pltpu.VMEM((_NROWS,), jnp.int32),
            pltpu.VMEM((_NROWS, _DIM), jnp.float32),
            pltpu.SemaphoreType.DMA,
            pltpu.SemaphoreType.DMA,
        ],
    )
    def bcast(flat_hbm, prompts_hbm, out_hbm, idx_v, rows_v, gsem, wsem):
        wid = lax.axis_index("s") * info.num_cores + lax.axis_index("c")
        base = wid * rpw
        pltpu.sync_copy(flat_hbm, idx_v)
        pltpu.async_copy(prompts_hbm.at[idx_v], rows_v, gsem).wait()
        copies = [pltpu.async_copy(rows_v, out_hbm.at[base + k], wsem)
                  for k in range(rpw)]
        for c in copies:
            c.wait()

    return bcast


_bcast_cache = []


def kernel(query, prompt_key, prompts, frequency):
    if not _bcast_cache:
        _bcast_cache.append(_make_bcast())
    sim, unsim, flat2d = _selector_call(query, prompt_key,
                                        frequency.reshape(1, _POOL))
    flat = flat2d.reshape(_NROWS)
    sel_big = _bcast_cache[0](flat, prompts.reshape(_POOL * _PLEN, _DIM))
    selection = sel_big.reshape(_B, _SEL, _PLEN, _DIM)
    return sim, unsim, selection


# speedup vs baseline: 3.1861x; 1.0209x over previous
import functools

import jax
import jax.numpy as jnp
from jax import lax
from jax.experimental import pallas as pl
from jax.experimental.pallas import tpu as pltpu
from jax.experimental.pallas import tpu_sc as plsc

_POOL = 20
_SEL = 10
_PLEN = 8
_DIM = 1024
_B = 1024


def _rowsum(s):
    acc = s[:, 0:128]
    for i in range(1, 8):
        acc = acc + s[:, i * 128:(i + 1) * 128]
    u = acc[:, 0:8]
    for k in range(1, 16):
        u = u + acc[:, 8 * k:8 * (k + 1)]
    u = u[:, 0:4] + u[:, 4:8]
    u = u[:, 0:2] + u[:, 2:4]
    return u[:, 0:1] + u[:, 1:2]


def _selector_body(q_ref, pk_ref, freq_ref, match_ref, pos_ref, flat_ref):
    q = q_ref[...]
    pk = pk_ref[...]
    freq = freq_ref[...]

    dot = lax.dot_general(
        q, pk, (((1,), (1,)), ((), ())),
        preferred_element_type=jnp.float32,
    )
    qn = jnp.sqrt(_rowsum(q * q))
    kn = jnp.sqrt(_rowsum(pk * pk)).reshape(1, _POOL)
    match = dot / jnp.maximum(qn * kn, 1e-8)

    fr = 1.0 / freq
    scale = fr / jnp.maximum(jnp.sum(jnp.abs(fr)), 1e-12)
    scores = match * scale

    cols = lax.broadcasted_iota(jnp.int32, (1, _POOL), 1)

    rank = jnp.zeros((_B, _POOL), jnp.int32)
    for j in range(_POOL):
        sj = scores[:, j:j + 1]
        beat = (sj > scores) | ((sj == scores) & (cols > j))
        rank = rank + beat.astype(jnp.int32)
    member = (rank < _SEL).astype(jnp.int32)
    counts = jnp.sum(member, axis=0, keepdims=True)
    masked = jnp.where(counts > 0, counts, -1)

    r_cols = []
    for p in range(_POOL):
        mp = masked[:, p:p + 1]
        beat = (masked > mp) | ((masked == mp) & (cols < p))
        r_cols.append(jnp.sum(beat.astype(jnp.int32), axis=1, keepdims=True))
    r = jnp.concatenate(r_cols, axis=1)
    selected = r < _SEL

    u_cols = []
    for p in range(_POOL):
        below = ((cols < p) & jnp.logical_not(selected)).astype(jnp.int32)
        u_cols.append(jnp.sum(below, axis=1, keepdims=True))
    u = jnp.concatenate(u_cols, axis=1)

    pos20 = jnp.where(selected, r, u + _SEL)
    match_ref[...] = match
    pos_ref[...] = pos20

    selrow_cols = []
    for j in range(_SEL):
        mj = (selected & (r == j))
        selrow_cols.append(jnp.sum(mj.astype(jnp.int32) * cols,
                                   axis=1, keepdims=True))

    flat_cols = []
    for j in range(_SEL):
        for l in range(_PLEN):
            flat_cols.append(selrow_cols[j] * _PLEN + l)
    flat_ref[...] = jnp.concatenate(flat_cols, axis=1)


def _gather_body(match_ref, pos_ref, sim_ref, unsim_ref):
    match = match_ref[...]
    pos20 = pos_ref[...]
    cols = lax.broadcasted_iota(jnp.int32, (1, _POOL), 1)
    y = jnp.zeros((_B, _POOL), jnp.float32)
    for p in range(_POOL):
        onehot = (pos20[:, p:p + 1] == cols).astype(jnp.float32)
        y = y + match[:, p:p + 1] * onehot
    sim_ref[...] = y[:, 0:_SEL]
    unsim_ref[...] = y[:, _SEL:_POOL]


@jax.jit
def _selector_call(query, prompt_key, freq2d):
    return pl.pallas_call(
        _selector_body,
        out_shape=(
            jax.ShapeDtypeStruct((_B, _POOL), jnp.float32),
            jax.ShapeDtypeStruct((1, _POOL), jnp.int32),
            jax.ShapeDtypeStruct((1, _SEL * _PLEN), jnp.int32),
        ),
    )(query, prompt_key, freq2d)


@jax.jit
def _gather_call(match, pos20):
    return pl.pallas_call(
        _gather_body,
        out_shape=(
            jax.ShapeDtypeStruct((_B, _SEL), jnp.float32),
            jax.ShapeDtypeStruct((_B, _SEL), jnp.float32),
        ),
    )(match, pos20)


_NROWS = _SEL * _PLEN


def _make_bcast():
    info = plsc.get_sparse_core_info()
    nw = info.num_cores * info.num_subcores
    rpw = _B // nw
    mesh = plsc.VectorSubcoreMesh(core_axis_name="c", subcore_axis_name="s")

    @functools.partial(
        pl.kernel,
        mesh=mesh,
        out_type=jax.ShapeDtypeStruct((_B, _NROWS, _DIM), jnp.float32),
        scratch_types=[
            pltpu.VMEM((_NROWS,), jnp.int32),
            pltpu.VMEM((_NROWS, _DIM), jnp.float32),
            pltpu.SemaphoreType.DMA,
            pltpu.SemaphoreType.DMA,
        ],
    )
    def bcast(flat_hbm, prompts_hbm, out_hbm, idx_v, rows_v, gsem, wsem):
        wid = lax.axis_index("s") * info.num_cores + lax.axis_index("c")
        base = wid * rpw
        pltpu.sync_copy(flat_hbm, idx_v)
        pltpu.async_copy(prompts_hbm.at[idx_v], rows_v, gsem).wait()
        copies = [pltpu.async_copy(rows_v, out_hbm.at[base + k], wsem)
                  for k in range(rpw)]
        for c in copies:
            c.wait()

    return bcast


_bcast_cache = []


def kernel(query, prompt_key, prompts, frequency):
    if not _bcast_cache:
        _bcast_cache.append(_make_bcast())
    match, pos20, flat2d = _selector_call(query, prompt_key,
                                          frequency.reshape(1, _POOL))
    flat = flat2d.reshape(_NROWS)
    sim, unsim = _gather_call(match, pos20)
    sel_big = _bcast_cache[0](flat, prompts.reshape(_POOL * _PLEN, _DIM))
    selection = sel_big.reshape(_B, _SEL, _PLEN, _DIM)
    return sim, unsim, selection
